# baseline (speedup 1.0000x reference)
import jax
import jax.numpy as jnp
from jax import lax
from jax.experimental import pallas as pl
from jax.experimental.pallas import tpu as pltpu

_LAMBDA = 25.0
_MU = 25.0
_NU = 1.0
_ALPHA = 0.25
_EPS = 1e-4
_K0, _K1 = 20, 4


def _local_stats_kernel(za_ref, zb_ref, lat_ref, lbt_ref, out_ref):
    for s in range(za_ref.shape[0]):
        _one_sample(za_ref[s], zb_ref[s], lat_ref[s], lbt_ref[s], out_ref, s)


def _one_sample(za, zb, lat, lbt, out_ref, s):
    n = za.shape[0]

    ones_c = jnp.ones((1, za.shape[1]), jnp.float32)

    na = jnp.sum(za * za, axis=1, keepdims=True)
    nb_t = lax.dot_general(ones_c, zb * zb, (((1,), (1,)), ((), ())),
                           preferred_element_type=jnp.float32)
    cross = lax.dot_general(za.astype(jnp.bfloat16), zb.astype(jnp.bfloat16),
                            (((1,), (1,)), ((), ())),
                            preferred_element_type=jnp.float32)
    d2f = na + nb_t - 2.0 * cross

    dx = jnp.transpose(lat[0:1, :]) - lbt[0:1, :]
    dy = jnp.transpose(lat[1:2, :]) - lbt[1:2, :]
    d2l = dx * dx + dy * dy

    rmf = jnp.min(d2f, axis=1, keepdims=True)
    cmf = jnp.min(d2f, axis=0, keepdims=True)
    rml = jnp.min(d2l, axis=1, keepdims=True)
    cml = jnp.min(d2l, axis=0, keepdims=True)
    ga = jnp.sum(jnp.where(d2l == rml, d2f, 0.0), axis=1, keepdims=True)
    gb = jnp.sum(jnp.where(d2l == cml, d2f, 0.0), axis=0, keepdims=True)

    packed = jnp.concatenate(
        [jnp.transpose(rmf), jnp.transpose(rml), jnp.transpose(ga),
         cmf, cml, gb, cmf, cml], axis=0)
    out_ref[s] = jnp.maximum(packed, 0.0)


def _ksmallest_sum(keys, k):
    total = jnp.zeros((keys.shape[0], 1), jnp.float32)
    for _ in range(k):
        m = jnp.min(keys, axis=1, keepdims=True)
        total = total + m
        keys = jnp.where(keys == m, jnp.float32(jnp.inf), keys)
    return jnp.sum(total)


def _topk_sum(keys, vals, k):
    total = jnp.zeros((keys.shape[0], 1), jnp.float32)
    for _ in range(k):
        m = jnp.min(keys, axis=1, keepdims=True)
        sel = keys == m
        total = total + jnp.sum(jnp.where(sel, vals, 0.0), axis=1,
                                keepdims=True)
        keys = jnp.where(sel, jnp.float32(jnp.inf), keys)
    return jnp.sum(total)


def _finalize_kernel(za_ref, zb_ref, rmf_ref, cmf_ref, rml_ref, cml_ref,
                     ga_ref, gb_ref, o_ref):
    za = za_ref[...]
    zb = zb_ref[...]
    b, d = za.shape
    bn = float(b)
    nm1 = bn - 1.0

    def global_stats(x):
        mean0 = jnp.mean(x, axis=0, keepdims=True)
        xc = x - mean0
        s = jnp.sum(xc * xc, axis=0, keepdims=True)
        g = lax.dot_general(xc, xc, (((1,), (1,)), ((), ())),
                            preferred_element_type=jnp.float32)
        cov = (jnp.sum(g * g) - jnp.sum(s * s)) / (nm1 * nm1 * d)
        std = jnp.sqrt(s / nm1 + _EPS)
        var = jnp.mean(jnp.maximum(1.0 - std, 0.0))
        return var, cov

    var_a, cov_a = global_stats(za)
    var_b, cov_b = global_stats(zb)
    diff = za - zb
    inv_g = jnp.mean(diff * diff)
    g_loss = (_LAMBDA * inv_g + _MU * 0.5 * (var_a + var_b)
              + _NU * (cov_a + cov_b))

    rmf = rmf_ref[...]
    cmf = cmf_ref[...]
    rml = rml_ref[...]
    cml = cml_ref[...]
    ga = ga_ref[...]
    gb = gb_ref[...]
    nb = rmf.shape[0]
    c = 768.0

    s0 = _ksmallest_sum(rmf, _K0)
    s1 = _ksmallest_sum(cmf, _K1)
    s2 = _topk_sum(rml, ga, _K0)
    s3 = _topk_sum(cml, gb, _K1)

    inv_l = (s0 / (2.0 * nb * _K0 * c) + s1 / (2.0 * nb * _K1 * c)
             + s2 / (2.0 * nb * _K0 * c) + s3 / (2.0 * nb * _K1 * c))
    l_loss = _LAMBDA * inv_l

    out = _ALPHA * g_loss + (1.0 - _ALPHA) * l_loss
    o_ref[...] = jnp.broadcast_to(out, (1, 1))


def kernel(z_a, z_b, z_a_local, z_b_local, location_a, location_b,
           interpret=False):
    b, h, w, c = z_a_local.shape
    n = h * w
    za = z_a_local.reshape(b, n, c)
    zb = z_b_local.reshape(b, n, c)
    lat = location_a.reshape(b, n, 2).transpose(0, 2, 1)
    lbt = location_b.reshape(b, n, 2).transpose(0, 2, 1)

    sb = 8
    packed = pl.pallas_call(
        _local_stats_kernel,
        grid=(b // sb,),
        in_specs=[pl.BlockSpec((sb, n, c), lambda i: (i, 0, 0)),
                  pl.BlockSpec((sb, n, c), lambda i: (i, 0, 0)),
                  pl.BlockSpec((sb, 2, n), lambda i: (i, 0, 0)),
                  pl.BlockSpec((sb, 2, n), lambda i: (i, 0, 0))],
        out_specs=pl.BlockSpec((sb, 8, n), lambda i: (i, 0, 0)),
        out_shape=jax.ShapeDtypeStruct((b, 8, n), jnp.float32),
        compiler_params=pltpu.CompilerParams(
            vmem_limit_bytes=128 * 1024 * 1024),
        interpret=interpret,
    )(za, zb, lat, lbt)

    out = pl.pallas_call(
        _finalize_kernel,
        out_shape=jax.ShapeDtypeStruct((1, 1), jnp.float32),
        interpret=interpret,
    )(z_a, z_b, packed[:, 0, :], packed[:, 3, :], packed[:, 1, :],
      packed[:, 4, :], packed[:, 2, :], packed[:, 5, :])
    return out.reshape(())


# speedup vs baseline: 1.0440x; 1.0440x over previous
import jax
import jax.numpy as jnp
from jax import lax
from jax.experimental import pallas as pl
from jax.experimental.pallas import tpu as pltpu

_LAMBDA = 25.0
_MU = 25.0
_NU = 1.0
_ALPHA = 0.25
_EPS = 1e-4
_K0, _K1 = 20, 4


def _local_stats_kernel(za_ref, zb_ref, lat_ref, lbt_ref, out_ref):
    for s in range(za_ref.shape[0]):
        _one_sample(za_ref[s], zb_ref[s], lat_ref[s], lbt_ref[s], out_ref, s)


def _one_sample(za, zb, lat, lbt, out_ref, s):
    n = za.shape[0]

    ones_c = jnp.ones((1, za.shape[1]), jnp.float32)

    na = jnp.sum(za * za, axis=1, keepdims=True)
    nb_t = lax.dot_general(ones_c, zb * zb, (((1,), (1,)), ((), ())),
                           preferred_element_type=jnp.float32)
    cross = lax.dot_general(za.astype(jnp.bfloat16), zb.astype(jnp.bfloat16),
                            (((1,), (1,)), ((), ())),
                            preferred_element_type=jnp.float32)
    d2f = na + nb_t - 2.0 * cross

    dx = jnp.transpose(lat[0:1, :]) - lbt[0:1, :]
    dy = jnp.transpose(lat[1:2, :]) - lbt[1:2, :]
    d2l = dx * dx + dy * dy

    rmf = jnp.min(d2f, axis=1, keepdims=True)
    cmf = jnp.min(d2f, axis=0, keepdims=True)
    rml = jnp.min(d2l, axis=1, keepdims=True)
    cml = jnp.min(d2l, axis=0, keepdims=True)
    ga = jnp.sum(jnp.where(d2l == rml, d2f, 0.0), axis=1, keepdims=True)
    gb = jnp.sum(jnp.where(d2l == cml, d2f, 0.0), axis=0, keepdims=True)

    packed = jnp.concatenate(
        [jnp.transpose(rmf), jnp.transpose(rml), jnp.transpose(ga),
         cmf, cml, gb, cmf, cml], axis=0)
    out_ref[s] = jnp.maximum(packed, 0.0)


def _ksmallest_sum(keys, k):
    total = jnp.zeros((keys.shape[0], 1), jnp.float32)
    for _ in range(k):
        m = jnp.min(keys, axis=1, keepdims=True)
        total = total + m
        keys = jnp.where(keys == m, jnp.float32(jnp.inf), keys)
    return jnp.sum(total)


def _topk_sum(keys, vals, k):
    total = jnp.zeros((keys.shape[0], 1), jnp.float32)
    for _ in range(k):
        m = jnp.min(keys, axis=1, keepdims=True)
        sel = keys == m
        total = total + jnp.sum(jnp.where(sel, vals, 0.0), axis=1,
                                keepdims=True)
        keys = jnp.where(sel, jnp.float32(jnp.inf), keys)
    return jnp.sum(total)


def _finalize_kernel(za_ref, zb_ref, rmf_ref, cmf_ref, rml_ref, cml_ref,
                     ga_ref, gb_ref, o_ref):
    za = za_ref[...]
    zb = zb_ref[...]
    b, d = za.shape
    bn = float(b)
    nm1 = bn - 1.0

    def global_stats(x):
        mean0 = jnp.mean(x, axis=0, keepdims=True)
        xc = x - mean0
        s = jnp.sum(xc * xc, axis=0, keepdims=True)
        g = lax.dot_general(xc, xc, (((1,), (1,)), ((), ())),
                            preferred_element_type=jnp.float32)
        cov = (jnp.sum(g * g) - jnp.sum(s * s)) / (nm1 * nm1 * d)
        std = jnp.sqrt(s / nm1 + _EPS)
        var = jnp.mean(jnp.maximum(1.0 - std, 0.0))
        return var, cov

    var_a, cov_a = global_stats(za)
    var_b, cov_b = global_stats(zb)
    diff = za - zb
    inv_g = jnp.mean(diff * diff)
    g_loss = (_LAMBDA * inv_g + _MU * 0.5 * (var_a + var_b)
              + _NU * (cov_a + cov_b))

    rmf = rmf_ref[...]
    cmf = cmf_ref[...]
    rml = rml_ref[...]
    cml = cml_ref[...]
    ga = ga_ref[...]
    gb = gb_ref[...]
    nb = rmf.shape[0]
    c = 768.0

    s0 = _ksmallest_sum(rmf, _K0)
    s1 = _ksmallest_sum(cmf, _K1)
    s2 = _topk_sum(rml, ga, _K0)
    s3 = _topk_sum(cml, gb, _K1)

    inv_l = (s0 / (2.0 * nb * _K0 * c) + s1 / (2.0 * nb * _K1 * c)
             + s2 / (2.0 * nb * _K0 * c) + s3 / (2.0 * nb * _K1 * c))
    l_loss = _LAMBDA * inv_l

    out = _ALPHA * g_loss + (1.0 - _ALPHA) * l_loss
    o_ref[...] = jnp.broadcast_to(out, (1, 1))


def kernel(z_a, z_b, z_a_local, z_b_local, location_a, location_b):
    b, h, w, c = z_a_local.shape
    n = h * w
    za = z_a_local.reshape(b, n, c)
    zb = z_b_local.reshape(b, n, c)
    lat = location_a.reshape(b, n, 2).transpose(0, 2, 1)
    lbt = location_b.reshape(b, n, 2).transpose(0, 2, 1)

    sb = 4
    packed = pl.pallas_call(
        _local_stats_kernel,
        grid=(b // sb,),
        in_specs=[pl.BlockSpec((sb, n, c), lambda i: (i, 0, 0)),
                  pl.BlockSpec((sb, n, c), lambda i: (i, 0, 0)),
                  pl.BlockSpec((sb, 2, n), lambda i: (i, 0, 0)),
                  pl.BlockSpec((sb, 2, n), lambda i: (i, 0, 0))],
        out_specs=pl.BlockSpec((sb, 8, n), lambda i: (i, 0, 0)),
        out_shape=jax.ShapeDtypeStruct((b, 8, n), jnp.float32),
        compiler_params=pltpu.CompilerParams(
            vmem_limit_bytes=128 * 1024 * 1024),
    )(za, zb, lat, lbt)

    out = pl.pallas_call(
        _finalize_kernel,
        out_shape=jax.ShapeDtypeStruct((1, 1), jnp.float32),
    )(z_a, z_b, packed[:, 0, :], packed[:, 3, :], packed[:, 1, :],
      packed[:, 4, :], packed[:, 2, :], packed[:, 5, :])
    return out.reshape(())
